# baseline (speedup 1.0000x reference)
import functools

import jax
import jax.numpy as jnp
from jax import lax
from jax.experimental import pallas as pl
from jax.experimental.pallas import tpu as pltpu
from jax.experimental.pallas import tpu_sc as plsc

_BATCH = 16384
_EMB = 16
_ROWS = 1000000
_LANE = 128
_BPAD = (-_ROWS) % _LANE
_C = 128


@functools.lru_cache(maxsize=None)
def _build_mf_kernel():
    info = plsc.get_sparse_core_info()
    nc, ns, nl = info.num_cores, info.num_subcores, info.num_lanes
    nw = nc * ns
    bpw = _BATCH // nw
    nrounds = bpw // _C
    nchunks = _C // nl
    pitch = nl + 1
    mesh = plsc.VectorSubcoreMesh(core_axis_name="c", subcore_axis_name="s")

    @functools.partial(
        pl.kernel,
        mesh=mesh,
        out_type=jax.ShapeDtypeStruct((_BATCH,), jnp.float32),
        compiler_params=pltpu.CompilerParams(needs_layout_passes=False),
        scratch_types=[
            pltpu.VMEM((bpw,), jnp.int32),
            pltpu.VMEM((bpw,), jnp.int32),
            pltpu.VMEM((_C,), jnp.int32),
            pltpu.VMEM((_C,), jnp.int32),
            pltpu.VMEM((_C,), jnp.int32),
            pltpu.VMEM((_C,), jnp.int32),
            pltpu.VMEM((_C, _LANE), jnp.float32),
            pltpu.VMEM((_C, _LANE), jnp.float32),
            pltpu.VMEM((_C, _LANE), jnp.float32),
            pltpu.VMEM((_C, _LANE), jnp.float32),
            pltpu.VMEM((nl,), jnp.float32),
            pltpu.VMEM((_EMB * (nl + 1),), jnp.float32),
            pltpu.VMEM((bpw,), jnp.float32),
            pltpu.SemaphoreType.DMA,
        ],
    )
    def mf(u_id, i_id, uemb, ubias, iemb, ibias, mean16, out,
           uidx_v, iidx_v, ues_i, ies_i, ubs_i, ibs_i,
           ue_slab, ie_slab, ub_slab, ib_slab, mean_v, pt_v, out_v, sem):
        wid = lax.axis_index("s") * nc + lax.axis_index("c")
        base = wid * bpw
        pltpu.sync_copy(u_id.at[pl.ds(base, bpw)], uidx_v)
        pltpu.sync_copy(i_id.at[pl.ds(base, bpw)], iidx_v)
        pltpu.sync_copy(mean16, mean_v)

        col = lax.iota(jnp.int32, nl) * pitch
        lanes = lax.iota(jnp.int32, nl)
        mean_vec = mean_v[...]

        def round_body(r, carry):
            rb = r * _C
            for k in range(nchunks):
                uv = uidx_v[pl.ds(rb + k * nl, nl)]
                iv = iidx_v[pl.ds(rb + k * nl, nl)]
                ues_i[pl.ds(k * nl, nl)] = lax.shift_right_logical(uv, 3)
                ies_i[pl.ds(k * nl, nl)] = lax.shift_right_logical(iv, 3)
                ubs_i[pl.ds(k * nl, nl)] = lax.shift_right_logical(uv, 7)
                ibs_i[pl.ds(k * nl, nl)] = lax.shift_right_logical(iv, 7)
            c1 = pltpu.async_copy(uemb.at[ues_i], ue_slab, sem)
            c2 = pltpu.async_copy(iemb.at[ies_i], ie_slab, sem)
            c3 = pltpu.async_copy(ubias.at[ubs_i], ub_slab, sem)
            c4 = pltpu.async_copy(ibias.at[ibs_i], ib_slab, sem)
            c1.wait()
            c2.wait()
            c3.wait()
            c4.wait()
            for k in range(nchunks):
                uv = uidx_v[pl.ds(rb + k * nl, nl)]
                iv = iidx_v[pl.ds(rb + k * nl, nl)]
                uoff = (uv & 7) * _EMB
                ioff = (iv & 7) * _EMB
                for l in range(nl):
                    urow = ue_slab[k * nl + l, pl.ds(uoff[l], _EMB)]
                    irow = ie_slab[k * nl + l, pl.ds(ioff[l], _EMB)]
                    plsc.store_scatter(pt_v, [col + l], urow * irow)
                rows = lanes + k * nl
                ubv = plsc.load_gather(ub_slab, [rows, uv & (_LANE - 1)])
                ibv = plsc.load_gather(ib_slab, [rows, iv & (_LANE - 1)])
                acc = ubv + ibv + mean_vec
                for d in range(_EMB):
                    acc = acc + pt_v[pl.ds(d * pitch, nl)]
                out_v[pl.ds(rb + k * nl, nl)] = 1.0 / (1.0 + jnp.exp(-acc))
            return carry

        lax.fori_loop(0, nrounds, round_body, 0)
        pltpu.sync_copy(out_v, out.at[pl.ds(base, bpw)])

    return mf


def kernel(data, user_emb, user_bias, item_emb, item_bias, mean):
    u_id = data[0].astype(jnp.int32)
    i_id = data[1].astype(jnp.int32)
    mean16 = jnp.broadcast_to(mean.astype(jnp.float32), (16,))
    uemb_s = user_emb.reshape(_ROWS * _EMB // _LANE, _LANE)
    iemb_s = item_emb.reshape(_ROWS * _EMB // _LANE, _LANE)
    ub_s = jnp.pad(user_bias.reshape(-1), (0, _BPAD)).reshape(-1, _LANE)
    ib_s = jnp.pad(item_bias.reshape(-1), (0, _BPAD)).reshape(-1, _LANE)
    mf = _build_mf_kernel()
    return mf(u_id, i_id, uemb_s, ub_s, iemb_s, ib_s, mean16)


# speedup vs baseline: 1.0167x; 1.0167x over previous
import functools

import jax
import jax.numpy as jnp
from jax import lax
from jax.experimental import pallas as pl
from jax.experimental.pallas import tpu as pltpu
from jax.experimental.pallas import tpu_sc as plsc

_BATCH = 16384
_EMB = 16


@functools.lru_cache(maxsize=None)
def _build_mf_kernel():
    info = plsc.get_sparse_core_info()
    nc, ns, nl = info.num_cores, info.num_subcores, info.num_lanes
    nw = nc * ns
    bpw = _BATCH // nw
    nchunks = bpw // nl
    pitch = nl + 1
    mesh = plsc.VectorSubcoreMesh(core_axis_name="c", subcore_axis_name="s")

    @functools.partial(
        pl.kernel,
        mesh=mesh,
        out_type=jax.ShapeDtypeStruct((_BATCH,), jnp.float32),
        compiler_params=pltpu.CompilerParams(
            needs_layout_passes=False, use_tc_tiling_on_sc=False),
        scratch_types=[
            pltpu.VMEM((bpw,), jnp.int32),
            pltpu.VMEM((bpw,), jnp.int32),
            pltpu.VMEM((bpw, _EMB), jnp.float32),
            pltpu.VMEM((bpw, _EMB), jnp.float32),
            pltpu.VMEM((bpw,), jnp.float32),
            pltpu.VMEM((bpw,), jnp.float32),
            pltpu.VMEM((nl,), jnp.float32),
            pltpu.VMEM((_EMB * (nl + 1),), jnp.float32),
            pltpu.VMEM((bpw,), jnp.float32),
            pltpu.SemaphoreType.DMA,
        ],
    )
    def mf(u_id, i_id, uemb, ubiasT, iemb, ibiasT, mean16, out,
           uidx_v, iidx_v, urows_v, irows_v, ub_v, ib_v, mean_v, pt_v,
           out_v, sem):
        wid = lax.axis_index("s") * nc + lax.axis_index("c")
        base = wid * bpw
        pltpu.sync_copy(u_id.at[pl.ds(base, bpw)], uidx_v)
        pltpu.sync_copy(i_id.at[pl.ds(base, bpw)], iidx_v)
        pltpu.sync_copy(mean16, mean_v)
        c1 = pltpu.async_copy(uemb.at[uidx_v], urows_v, sem)
        c2 = pltpu.async_copy(iemb.at[iidx_v], irows_v, sem)
        c3 = pltpu.async_copy(ubiasT.at[0].at[uidx_v], ub_v, sem)
        c4 = pltpu.async_copy(ibiasT.at[0].at[iidx_v], ib_v, sem)
        c1.wait()
        c2.wait()
        c3.wait()
        c4.wait()

        col = lax.iota(jnp.int32, nl) * pitch
        mean_vec = mean_v[...]

        def chunk(c, carry):
            for l in range(nl):
                p = c * nl + l
                prod = urows_v[p, :] * irows_v[p, :]
                plsc.store_scatter(pt_v, [col + l], prod)
            acc = ub_v[pl.ds(c * nl, nl)] + ib_v[pl.ds(c * nl, nl)] + mean_vec
            for d in range(_EMB):
                acc = acc + pt_v[pl.ds(d * pitch, nl)]
            out_v[pl.ds(c * nl, nl)] = 1.0 / (1.0 + jnp.exp(-acc))
            return carry

        lax.fori_loop(0, nchunks, chunk, 0)
        pltpu.sync_copy(out_v, out.at[pl.ds(base, bpw)])

    return mf


def kernel(data, user_emb, user_bias, item_emb, item_bias, mean):
    u_id = data[0].astype(jnp.int32)
    i_id = data[1].astype(jnp.int32)
    mean16 = jnp.broadcast_to(mean.astype(jnp.float32), (16,))
    mf = _build_mf_kernel()
    return mf(u_id, i_id, user_emb, user_bias.T, item_emb, item_bias.T,
              mean16)


# speedup vs baseline: 5.6150x; 5.5229x over previous
import functools

import jax
import jax.numpy as jnp
from jax import lax
from jax.experimental import pallas as pl
from jax.experimental.pallas import tpu as pltpu
from jax.experimental.pallas import tpu_sc as plsc

_BATCH = 16384
_EMB = 16
_LANE = 128


@functools.lru_cache(maxsize=None)
def _build_mf_kernel():
    info = plsc.get_sparse_core_info()
    nc, ns, nl = info.num_cores, info.num_subcores, info.num_lanes
    nw = nc * ns
    bpw = _BATCH // nw
    nchunks = bpw // nl
    pitch = nl + 1
    mesh = plsc.VectorSubcoreMesh(core_axis_name="c", subcore_axis_name="s")

    @functools.partial(
        pl.kernel,
        mesh=mesh,
        out_type=jax.ShapeDtypeStruct((_BATCH,), jnp.float32),
        compiler_params=pltpu.CompilerParams(needs_layout_passes=False),
        scratch_types=[
            pltpu.VMEM((bpw,), jnp.int32),
            pltpu.VMEM((bpw,), jnp.int32),
            pltpu.VMEM((nl, _EMB, _LANE), jnp.float32),
            pltpu.VMEM((nl, _EMB, _LANE), jnp.float32),
            pltpu.VMEM((nl, 1, _LANE), jnp.float32),
            pltpu.VMEM((nl, 1, _LANE), jnp.float32),
            pltpu.VMEM((nl,), jnp.float32),
            pltpu.VMEM((_EMB * (nl + 1),), jnp.float32),
            pltpu.VMEM((bpw,), jnp.float32),
            pltpu.SemaphoreType.DMA,
        ],
    )
    def mf(u_id, i_id, uembT, ubiasT, iembT, ibiasT, mean16, out,
           uidx_v, iidx_v, ublk, iblk, ubb, ibb, mean_v, pt_v, out_v, sem):
        wid = lax.axis_index("s") * nc + lax.axis_index("c")
        base = wid * bpw
        pltpu.sync_copy(u_id.at[pl.ds(base, bpw)], uidx_v)
        pltpu.sync_copy(i_id.at[pl.ds(base, bpw)], iidx_v)
        pltpu.sync_copy(mean16, mean_v)

        lanes = lax.iota(jnp.int32, nl)
        col = lanes * pitch
        mean_vec = mean_v[...]

        def chunk(c, carry):
            uv = uidx_v[pl.ds(c * nl, nl)]
            iv = iidx_v[pl.ds(c * nl, nl)]
            ucol = uv & (_LANE - 1)
            icol = iv & (_LANE - 1)
            ualn = uv - ucol
            ialn = iv - icol
            copies = []
            for l in range(nl):
                ua = pl.multiple_of(ualn[l], _LANE)
                ia = pl.multiple_of(ialn[l], _LANE)
                copies.append(pltpu.async_copy(
                    uembT.at[:, pl.ds(ua, _LANE)], ublk.at[l], sem))
                copies.append(pltpu.async_copy(
                    iembT.at[:, pl.ds(ia, _LANE)], iblk.at[l], sem))
                copies.append(pltpu.async_copy(
                    ubiasT.at[:, pl.ds(ua, _LANE)], ubb.at[l], sem))
                copies.append(pltpu.async_copy(
                    ibiasT.at[:, pl.ds(ia, _LANE)], ibb.at[l], sem))
            for cp in copies:
                cp.wait()
            zer = jnp.zeros((nl,), jnp.int32)
            ubv = plsc.load_gather(ubb, [lanes, zer, ucol])
            ibv = plsc.load_gather(ibb, [lanes, zer, icol])
            acc = ubv + ibv + mean_vec
            for l in range(nl):
                lv = jnp.full((nl,), l, jnp.int32)
                uc = plsc.load_gather(ublk, [lv, lanes, zer + ucol[l]])
                ic = plsc.load_gather(iblk, [lv, lanes, zer + icol[l]])
                plsc.store_scatter(pt_v, [col + l], uc * ic)
            for d in range(_EMB):
                acc = acc + pt_v[pl.ds(d * pitch, nl)]
            out_v[pl.ds(c * nl, nl)] = 1.0 / (1.0 + jnp.exp(-acc))
            return carry

        lax.fori_loop(0, nchunks, chunk, 0)
        pltpu.sync_copy(out_v, out.at[pl.ds(base, bpw)])

    return mf


def kernel(data, user_emb, user_bias, item_emb, item_bias, mean):
    u_id = data[0].astype(jnp.int32)
    i_id = data[1].astype(jnp.int32)
    mean16 = jnp.broadcast_to(mean.astype(jnp.float32), (16,))
    mf = _build_mf_kernel()
    return mf(u_id, i_id, user_emb.T, user_bias.T, item_emb.T, item_bias.T,
              mean16)
